# baseline (speedup 1.0000x reference)
import functools

import jax
import jax.numpy as jnp
from jax import lax
from jax.experimental import pallas as pl
from jax.experimental.pallas import tpu as pltpu
from jax.experimental.pallas import tpu_sc as plsc

F32 = jnp.float32

NCORES = 2
NSUB = 16
KB = 4
CHUNK = KB * 128

BLK = 2048


def _sc_degree(cols2d, acc_rows):
    nchunk = cols2d.shape[0] // (NCORES * NSUB * KB)
    npair = nchunk // 2
    rpt = acc_rows // NSUB
    mesh = plsc.VectorSubcoreMesh(core_axis_name="c", subcore_axis_name="s")

    @functools.partial(
        pl.kernel,
        out_type=jax.ShapeDtypeStruct((NCORES * acc_rows,), F32),
        mesh=mesh,
        scratch_types=[
            pltpu.VMEM((2, KB, 128), jnp.int32),
            pltpu.VMEM((128,), F32),
            pltpu.VMEM((rpt,), F32),
            pltpu.VMEM_SHARED((acc_rows,), F32),
            pltpu.SemaphoreType.DMA,
            pltpu.SemaphoreType.DMA,
        ],
    )
    def deg_kernel(cols_hbm, cnt_hbm, cidx_v, ones_v, zd_v, deg_sh, s0, s1):
        cid = lax.axis_index("c")
        sid = lax.axis_index("s")
        w = cid * NSUB + sid
        tile_base = w * (nchunk * KB)
        sems = (s0, s1)
        for q in range(8):
            ones_v[pl.ds(q * 16, 16)] = jnp.full((16,), 1.0, F32)

        def zstore(i, carry):
            zd_v[pl.ds(i * 16, 16)] = jnp.zeros((16,), F32)
            return carry

        lax.fori_loop(0, rpt // 16, zstore, 0)
        pltpu.sync_copy(zd_v, deg_sh.at[pl.ds(sid * rpt, rpt)])
        plsc.subcore_barrier()

        def pair(t, carry):
            for p in range(2):
                g = 2 * t + p

                @pl.when(t > 0)
                def _drain():
                    for j in range(KB):
                        pltpu.make_async_copy(
                            ones_v, deg_sh.at[cidx_v.at[p, j]], sems[p]).wait()

                pltpu.sync_copy(cols_hbm.at[pl.ds(tile_base + g * KB, KB), :],
                                cidx_v.at[p])
                for j in range(KB):
                    pltpu.async_copy(ones_v, deg_sh.at[cidx_v.at[p, j]],
                                     sems[p], add=True)
            return carry

        lax.fori_loop(0, npair, pair, 0)
        for p in range(2):
            for j in range(KB):
                pltpu.make_async_copy(ones_v, deg_sh.at[cidx_v.at[p, j]],
                                      sems[p]).wait()
        plsc.subcore_barrier()
        pltpu.sync_copy(deg_sh.at[pl.ds(sid * rpt, rpt)], zd_v)
        pltpu.sync_copy(zd_v, cnt_hbm.at[pl.ds(cid * acc_rows + sid * rpt, rpt)])

    return deg_kernel(cols2d).reshape(NCORES, acc_rows)


def _sc_scatter(h2d, rows2d, cols2d, acc_rows, n):
    nchunk = cols2d.shape[0] // (NSUB * KB)
    npair = nchunk // 2
    rpt = acc_rows // NSUB
    zrows = rpt // 16
    mesh = plsc.VectorSubcoreMesh(core_axis_name="c", subcore_axis_name="s")

    @functools.partial(
        pl.kernel,
        out_type=jax.ShapeDtypeStruct((NCORES, acc_rows, 16), F32),
        mesh=mesh,
        scratch_types=[
            pltpu.VMEM((2, KB, 128), jnp.int32),
            pltpu.VMEM((2, KB, 128), jnp.int32),
            pltpu.VMEM((2, KB, 128, 16), F32),
            pltpu.VMEM((zrows, 16), F32),
            pltpu.VMEM_SHARED((acc_rows, 16), F32),
            pltpu.SemaphoreType.DMA,
            pltpu.SemaphoreType.DMA,
            pltpu.SemaphoreType.DMA,
            pltpu.SemaphoreType.DMA,
        ],
        compiler_params=pltpu.CompilerParams(use_tc_tiling_on_sc=False),
    )
    def acc_kernel(h_hbm, rows_hbm, cols_hbm, acc_hbm,
                   ridx_v, cidx_v, rows_v, za_v, acc_sh, g0, g1, t0, t1):
        cid = lax.axis_index("c")
        sid = lax.axis_index("s")
        off = cid * n
        tile_base = sid * (nchunk * KB)
        sem_g = (g0, g1)
        sem_s = (t0, t1)

        def zstore(i, carry):
            za_v[i, :] = jnp.zeros((16,), F32)
            return carry

        lax.fori_loop(0, zrows, zstore, 0)
        for q in range(16):
            pltpu.sync_copy(za_v,
                            acc_sh.at[pl.ds(sid * rpt + q * zrows, zrows), :])
        plsc.subcore_barrier()

        def load_and_gather(g, p):
            pltpu.sync_copy(rows_hbm.at[pl.ds(tile_base + g * KB, KB), :],
                            ridx_v.at[p])
            pltpu.sync_copy(cols_hbm.at[pl.ds(tile_base + g * KB, KB), :],
                            cidx_v.at[p])
            for j in range(KB):
                for q in range(8):
                    sl = ridx_v[p, j, pl.ds(q * 16, 16)]
                    ridx_v[p, j, pl.ds(q * 16, 16)] = sl + off
            for j in range(KB):
                pltpu.async_copy(h_hbm.at[ridx_v.at[p, j]], rows_v.at[p, j],
                                 sem_g[p])

        load_and_gather(0, 0)

        def pair(t, carry):
            for p in range(2):
                g = 2 * t + p
                for j in range(KB):
                    pltpu.make_async_copy(h_hbm.at[ridx_v.at[p, j]],
                                          rows_v.at[p, j], sem_g[p]).wait()
                for j in range(KB):
                    pltpu.async_copy(rows_v.at[p, j],
                                     acc_sh.at[cidx_v.at[p, j]], sem_s[p],
                                     add=True)
                op = 1 - p

                @pl.when(g > 0)
                def _drain():
                    for j in range(KB):
                        pltpu.make_async_copy(rows_v.at[op, j],
                                              acc_sh.at[cidx_v.at[op, j]],
                                              sem_s[op]).wait()

                @pl.when(g + 1 < nchunk)
                def _next():
                    load_and_gather(g + 1, op)
            return carry

        lax.fori_loop(0, npair, pair, 0)
        lastp = (nchunk - 1) % 2
        for j in range(KB):
            pltpu.make_async_copy(rows_v.at[lastp, j],
                                  acc_sh.at[cidx_v.at[lastp, j]],
                                  sem_s[lastp]).wait()
        plsc.subcore_barrier()
        for q in range(16):
            pltpu.sync_copy(acc_sh.at[pl.ds(sid * rpt + q * zrows, zrows), :],
                            za_v)
            pltpu.sync_copy(za_v,
                            acc_hbm.at[cid, pl.ds(sid * rpt + q * zrows, zrows), :])

    return acc_kernel(h2d, rows2d, cols2d)


def _tc_embed(x, W_conv, cnt):
    n, d_in = x.shape
    grid = (pl.cdiv(n, BLK),)

    def body(x_ref, w_ref, cnt_ref, h_ref, dinv_ref):
        deg = cnt_ref[0, :] + cnt_ref[1, :] + 1.0
        dinv = lax.rsqrt(deg)
        h = jnp.dot(x_ref[...], w_ref[...], preferred_element_type=F32)
        hp = h * dinv[:, None]
        h_ref[0] = hp[:, :16]
        h_ref[1] = hp[:, 16:]
        dinv_ref[...] = dinv[:, None]

    return pl.pallas_call(
        body,
        grid=grid,
        in_specs=[
            pl.BlockSpec((BLK, d_in), lambda i: (i, 0)),
            pl.BlockSpec(W_conv.shape, lambda i: (0, 0)),
            pl.BlockSpec((2, BLK), lambda i: (0, i)),
        ],
        out_specs=[
            pl.BlockSpec((2, BLK, 16), lambda i: (0, i, 0)),
            pl.BlockSpec((BLK, 1), lambda i: (i, 0)),
        ],
        out_shape=[
            jax.ShapeDtypeStruct((2, n, 16), F32),
            jax.ShapeDtypeStruct((n, 1), F32),
        ],
    )(x, W_conv, cnt)


def _tc_head(acc, h, dinv, b_conv, W1p, b1p, W2p, b2p):
    n = h.shape[1]
    d_out = W2p.shape[1]
    grid = (pl.cdiv(n, BLK),)

    def body(acc_ref, h_ref, dinv_ref, bc_ref, w1_ref, b1_ref, w2_ref, b2_ref,
             out_ref):
        lo = acc_ref[0] + h_ref[0]
        hi = acc_ref[1] + h_ref[1]
        agg = jnp.concatenate([lo, hi], axis=1) * dinv_ref[...]
        h2 = jnp.maximum(agg + bc_ref[...], 0.0)
        h3 = jnp.maximum(
            jnp.dot(h2, w1_ref[...], preferred_element_type=F32) + b1_ref[...],
            0.0)
        out_ref[...] = (jnp.dot(h3, w2_ref[...], preferred_element_type=F32)
                        + b2_ref[...])

    return pl.pallas_call(
        body,
        grid=grid,
        in_specs=[
            pl.BlockSpec((2, BLK, 16), lambda i: (0, i, 0)),
            pl.BlockSpec((2, BLK, 16), lambda i: (0, i, 0)),
            pl.BlockSpec((BLK, 1), lambda i: (i, 0)),
            pl.BlockSpec(b_conv.shape, lambda i: (0, 0)),
            pl.BlockSpec(W1p.shape, lambda i: (0, 0)),
            pl.BlockSpec(b1p.shape, lambda i: (0, 0)),
            pl.BlockSpec(W2p.shape, lambda i: (0, 0)),
            pl.BlockSpec(b2p.shape, lambda i: (0, 0)),
        ],
        out_specs=pl.BlockSpec((BLK, d_out), lambda i: (i, 0)),
        out_shape=jax.ShapeDtypeStruct((n, d_out), F32),
    )(acc, h, dinv, b_conv, W1p, b1p, W2p, b2p)


def kernel(x, edge_index, W_conv, b_conv, W1, b1, W2, b2):
    n = x.shape[0]
    e = edge_index.shape[1]
    d1 = W_conv.shape[1]
    d2 = W1.shape[1]
    d_out = W2.shape[1]

    unit = NCORES * NSUB * CHUNK * 2
    ep = ((e + unit - 1) // unit) * unit
    acc_rows = ((n + 1 + 1023) // 1024) * 1024

    row_p = jnp.concatenate([edge_index[0],
                             jnp.zeros((ep - e,), jnp.int32)])
    col_p = jnp.concatenate([edge_index[1],
                             jnp.full((ep - e,), n, jnp.int32)])
    rows2d = row_p.reshape(ep // 128, 128)
    cols2d = col_p.reshape(ep // 128, 128)

    W1p = jnp.pad(W1, ((0, 0), (0, 128 - d2)))
    b1p = jnp.pad(b1, (0, 128 - d2)).reshape(1, 128)
    W2p = jnp.pad(W2, ((0, 128 - d2), (0, 0)))
    b2p = b2.reshape(1, d_out)
    bc = b_conv.reshape(1, d1)

    cnt = _sc_degree(cols2d, acc_rows)
    h, dinv = _tc_embed(x, W_conv, cnt)
    h2d = h.reshape(NCORES * n, 16)
    acc = _sc_scatter(h2d, rows2d, cols2d, acc_rows, n)
    out = _tc_head(acc, h, dinv, bc, W1p, b1p, W2p, b2p)
    return out


# speedup vs baseline: 1.1648x; 1.1648x over previous
import functools

import jax
import jax.numpy as jnp
from jax import lax
from jax.experimental import pallas as pl
from jax.experimental.pallas import tpu as pltpu
from jax.experimental.pallas import tpu_sc as plsc

F32 = jnp.float32

NCORES = 2
NSUB = 16
KB = 6
CHUNK = KB * 128

BLK = 2048


def _sc_degree(cols2d, acc_rows):
    nchunk = cols2d.shape[0] // (NCORES * NSUB * KB)
    npair = nchunk // 2
    rpt = acc_rows // NSUB
    mesh = plsc.VectorSubcoreMesh(core_axis_name="c", subcore_axis_name="s")

    @functools.partial(
        pl.kernel,
        out_type=jax.ShapeDtypeStruct((NCORES * acc_rows,), F32),
        mesh=mesh,
        scratch_types=[
            pltpu.VMEM((2, KB, 128), jnp.int32),
            pltpu.VMEM((128,), F32),
            pltpu.VMEM((rpt,), F32),
            pltpu.VMEM_SHARED((acc_rows,), F32),
            pltpu.SemaphoreType.DMA,
            pltpu.SemaphoreType.DMA,
        ],
        compiler_params=pltpu.CompilerParams(use_tc_tiling_on_sc=False),
    )
    def deg_kernel(cols_hbm, cnt_hbm, cidx_v, ones_v, zd_v, deg_sh, s0, s1):
        cid = lax.axis_index("c")
        sid = lax.axis_index("s")
        w = cid * NSUB + sid
        tile_base = w * (nchunk * KB)
        sems = (s0, s1)
        for q in range(8):
            ones_v[pl.ds(q * 16, 16)] = jnp.full((16,), 1.0, F32)

        def zstore(i, carry):
            zd_v[pl.ds(i * 16, 16)] = jnp.zeros((16,), F32)
            return carry

        lax.fori_loop(0, rpt // 16, zstore, 0)
        pltpu.sync_copy(zd_v, deg_sh.at[pl.ds(sid * rpt, rpt)])
        plsc.subcore_barrier()

        def pair(t, carry):
            for p in range(2):
                g = 2 * t + p

                @pl.when(t > 0)
                def _drain():
                    for j in range(KB):
                        pltpu.make_async_copy(
                            ones_v, deg_sh.at[cidx_v.at[p, j]], sems[p]).wait()

                pltpu.sync_copy(cols_hbm.at[pl.ds(tile_base + g * KB, KB), :],
                                cidx_v.at[p])
                for j in range(KB):
                    pltpu.async_copy(ones_v, deg_sh.at[cidx_v.at[p, j]],
                                     sems[p], add=True)
            return carry

        lax.fori_loop(0, npair, pair, 0)
        for p in range(2):
            for j in range(KB):
                pltpu.make_async_copy(ones_v, deg_sh.at[cidx_v.at[p, j]],
                                      sems[p]).wait()
        plsc.subcore_barrier()
        pltpu.sync_copy(deg_sh.at[pl.ds(sid * rpt, rpt)], zd_v)
        pltpu.sync_copy(zd_v, cnt_hbm.at[pl.ds(cid * acc_rows + sid * rpt, rpt)])

    return deg_kernel(cols2d).reshape(NCORES, acc_rows)


def _sc_scatter(h2d, rows2d, cols2d, acc_rows, n):
    nchunk = cols2d.shape[0] // (NSUB * KB)
    npair = nchunk // 2
    rpt = acc_rows // NSUB
    nz = rpt // 128
    mesh = plsc.VectorSubcoreMesh(core_axis_name="c", subcore_axis_name="s")

    @functools.partial(
        pl.kernel,
        out_type=jax.ShapeDtypeStruct((NCORES, acc_rows, 16), F32),
        mesh=mesh,
        scratch_types=[
            pltpu.VMEM((2, KB, 128), jnp.int32),
            pltpu.VMEM((2, KB, 128), jnp.int32),
            pltpu.VMEM((2, KB, 128, 16), F32),
            pltpu.VMEM_SHARED((acc_rows, 16), F32),
            pltpu.SemaphoreType.DMA,
            pltpu.SemaphoreType.DMA,
            pltpu.SemaphoreType.DMA,
            pltpu.SemaphoreType.DMA,
        ],
        compiler_params=pltpu.CompilerParams(use_tc_tiling_on_sc=False),
    )
    def acc_kernel(h_hbm, rows_hbm, cols_hbm, acc_hbm,
                   ridx_v, cidx_v, rows_v, acc_sh, g0, g1, t0, t1):
        cid = lax.axis_index("c")
        sid = lax.axis_index("s")
        off = cid * n
        tile_base = sid * (nchunk * KB)
        sem_g = (g0, g1)
        sem_s = (t0, t1)

        def zstore(i, carry):
            rows_v[0, 0, i, :] = jnp.zeros((16,), F32)
            return carry

        lax.fori_loop(0, 128, zstore, 0)
        for q in range(nz):
            pltpu.sync_copy(rows_v.at[0, 0],
                            acc_sh.at[pl.ds(sid * rpt + q * 128, 128), :])
        plsc.subcore_barrier()

        def load_and_gather(g, p):
            pltpu.sync_copy(rows_hbm.at[pl.ds(tile_base + g * KB, KB), :],
                            ridx_v.at[p])
            pltpu.sync_copy(cols_hbm.at[pl.ds(tile_base + g * KB, KB), :],
                            cidx_v.at[p])
            for j in range(KB):
                for q in range(8):
                    sl = ridx_v[p, j, pl.ds(q * 16, 16)]
                    ridx_v[p, j, pl.ds(q * 16, 16)] = sl + off
            for j in range(KB):
                pltpu.async_copy(h_hbm.at[ridx_v.at[p, j]], rows_v.at[p, j],
                                 sem_g[p])

        load_and_gather(0, 0)

        def pair(t, carry):
            for p in range(2):
                g = 2 * t + p
                op = 1 - p

                @pl.when(g > 0)
                def _drain():
                    for j in range(KB):
                        pltpu.make_async_copy(rows_v.at[op, j],
                                              acc_sh.at[cidx_v.at[op, j]],
                                              sem_s[op]).wait()

                @pl.when(g + 1 < nchunk)
                def _next():
                    load_and_gather(g + 1, op)

                for j in range(KB):
                    pltpu.make_async_copy(h_hbm.at[ridx_v.at[p, j]],
                                          rows_v.at[p, j], sem_g[p]).wait()
                for j in range(KB):
                    pltpu.async_copy(rows_v.at[p, j],
                                     acc_sh.at[cidx_v.at[p, j]], sem_s[p],
                                     add=True)
            return carry

        lax.fori_loop(0, npair, pair, 0)
        lastp = (nchunk - 1) % 2
        for j in range(KB):
            pltpu.make_async_copy(rows_v.at[lastp, j],
                                  acc_sh.at[cidx_v.at[lastp, j]],
                                  sem_s[lastp]).wait()
        plsc.subcore_barrier()
        for q in range(nz):
            pltpu.sync_copy(acc_sh.at[pl.ds(sid * rpt + q * 128, 128), :],
                            rows_v.at[0, 0])
            pltpu.sync_copy(rows_v.at[0, 0],
                            acc_hbm.at[cid, pl.ds(sid * rpt + q * 128, 128), :])

    return acc_kernel(h2d, rows2d, cols2d)


def _tc_embed(x, W_conv, cnt):
    n, d_in = x.shape
    grid = (pl.cdiv(n, BLK),)

    def body(x_ref, w_ref, cnt_ref, h_ref, dinv_ref):
        deg = cnt_ref[0, :] + cnt_ref[1, :] + 1.0
        dinv = lax.rsqrt(deg)
        h = jnp.dot(x_ref[...], w_ref[...], preferred_element_type=F32)
        hp = h * dinv[:, None]
        h_ref[0] = hp[:, :16]
        h_ref[1] = hp[:, 16:]
        dinv_ref[...] = dinv[:, None]

    return pl.pallas_call(
        body,
        grid=grid,
        in_specs=[
            pl.BlockSpec((BLK, d_in), lambda i: (i, 0)),
            pl.BlockSpec(W_conv.shape, lambda i: (0, 0)),
            pl.BlockSpec((2, BLK), lambda i: (0, i)),
        ],
        out_specs=[
            pl.BlockSpec((2, BLK, 16), lambda i: (0, i, 0)),
            pl.BlockSpec((BLK, 1), lambda i: (i, 0)),
        ],
        out_shape=[
            jax.ShapeDtypeStruct((2, n, 16), F32),
            jax.ShapeDtypeStruct((n, 1), F32),
        ],
    )(x, W_conv, cnt)


def _tc_head(acc, h, dinv, b_conv, W1p, b1p, W2p, b2p):
    n = h.shape[1]
    d_out = W2p.shape[1]
    grid = (pl.cdiv(n, BLK),)

    def body(acc_ref, h_ref, dinv_ref, bc_ref, w1_ref, b1_ref, w2_ref, b2_ref,
             out_ref):
        lo = acc_ref[0] + h_ref[0]
        hi = acc_ref[1] + h_ref[1]
        agg = jnp.concatenate([lo, hi], axis=1) * dinv_ref[...]
        h2 = jnp.maximum(agg + bc_ref[...], 0.0)
        h3 = jnp.maximum(
            jnp.dot(h2, w1_ref[...], preferred_element_type=F32) + b1_ref[...],
            0.0)
        out_ref[...] = (jnp.dot(h3, w2_ref[...], preferred_element_type=F32)
                        + b2_ref[...])

    return pl.pallas_call(
        body,
        grid=grid,
        in_specs=[
            pl.BlockSpec((2, BLK, 16), lambda i: (0, i, 0)),
            pl.BlockSpec((2, BLK, 16), lambda i: (0, i, 0)),
            pl.BlockSpec((BLK, 1), lambda i: (i, 0)),
            pl.BlockSpec(b_conv.shape, lambda i: (0, 0)),
            pl.BlockSpec(W1p.shape, lambda i: (0, 0)),
            pl.BlockSpec(b1p.shape, lambda i: (0, 0)),
            pl.BlockSpec(W2p.shape, lambda i: (0, 0)),
            pl.BlockSpec(b2p.shape, lambda i: (0, 0)),
        ],
        out_specs=pl.BlockSpec((BLK, d_out), lambda i: (i, 0)),
        out_shape=jax.ShapeDtypeStruct((n, d_out), F32),
    )(acc, h, dinv, b_conv, W1p, b1p, W2p, b2p)


def kernel(x, edge_index, W_conv, b_conv, W1, b1, W2, b2):
    n = x.shape[0]
    e = edge_index.shape[1]
    d1 = W_conv.shape[1]
    d2 = W1.shape[1]
    d_out = W2.shape[1]

    unit = NCORES * NSUB * CHUNK * 2
    ep = ((e + unit - 1) // unit) * unit
    acc_rows = ((n + 1 + 1023) // 1024) * 1024

    row_p = jnp.concatenate([edge_index[0],
                             jnp.zeros((ep - e,), jnp.int32)])
    col_p = jnp.concatenate([edge_index[1],
                             jnp.full((ep - e,), n, jnp.int32)])
    rows2d = row_p.reshape(ep // 128, 128)
    cols2d = col_p.reshape(ep // 128, 128)

    W1p = jnp.pad(W1, ((0, 0), (0, 128 - d2)))
    b1p = jnp.pad(b1, (0, 128 - d2)).reshape(1, 128)
    W2p = jnp.pad(W2, ((0, 128 - d2), (0, 0)))
    b2p = b2.reshape(1, d_out)
    bc = b_conv.reshape(1, d1)

    cnt = _sc_degree(cols2d, acc_rows)
    h, dinv = _tc_embed(x, W_conv, cnt)
    h2d = h.reshape(NCORES * n, 16)
    acc = _sc_scatter(h2d, rows2d, cols2d, acc_rows, n)
    out = _tc_head(acc, h, dinv, bc, W1p, b1p, W2p, b2p)
    return out


# speedup vs baseline: 1.1938x; 1.0249x over previous
import functools

import jax
import jax.numpy as jnp
from jax import lax
from jax.experimental import pallas as pl
from jax.experimental.pallas import tpu as pltpu
from jax.experimental.pallas import tpu_sc as plsc

F32 = jnp.float32

NCORES = 2
NSUB = 16
KB = 6
CHUNK = KB * 128

BLK = 4096


def _sc_degree(cols2d, acc_rows):
    nchunk = cols2d.shape[0] // (NCORES * NSUB * KB)
    npair = nchunk // 2
    rpt = acc_rows // NSUB
    mesh = plsc.VectorSubcoreMesh(core_axis_name="c", subcore_axis_name="s")

    @functools.partial(
        pl.kernel,
        out_type=jax.ShapeDtypeStruct((NCORES * acc_rows,), F32),
        mesh=mesh,
        scratch_types=[
            pltpu.VMEM((2, KB, 128), jnp.int32),
            pltpu.VMEM((128,), F32),
            pltpu.VMEM((rpt,), F32),
            pltpu.VMEM_SHARED((acc_rows,), F32),
            pltpu.SemaphoreType.DMA,
            pltpu.SemaphoreType.DMA,
        ],
        compiler_params=pltpu.CompilerParams(use_tc_tiling_on_sc=False),
    )
    def deg_kernel(cols_hbm, cnt_hbm, cidx_v, ones_v, zd_v, deg_sh, s0, s1):
        cid = lax.axis_index("c")
        sid = lax.axis_index("s")
        w = cid * NSUB + sid
        tile_base = w * (nchunk * KB)
        sems = (s0, s1)
        for q in range(8):
            ones_v[pl.ds(q * 16, 16)] = jnp.full((16,), 1.0, F32)

        def zstore(i, carry):
            zd_v[pl.ds(i * 16, 16)] = jnp.zeros((16,), F32)
            return carry

        lax.fori_loop(0, rpt // 16, zstore, 0)
        pltpu.sync_copy(zd_v, deg_sh.at[pl.ds(sid * rpt, rpt)])
        plsc.subcore_barrier()

        def pair(t, carry):
            for p in range(2):
                g = 2 * t + p

                @pl.when(t > 0)
                def _drain():
                    for j in range(KB):
                        pltpu.make_async_copy(
                            ones_v, deg_sh.at[cidx_v.at[p, j]], sems[p]).wait()

                pltpu.sync_copy(cols_hbm.at[pl.ds(tile_base + g * KB, KB), :],
                                cidx_v.at[p])
                for j in range(KB):
                    pltpu.async_copy(ones_v, deg_sh.at[cidx_v.at[p, j]],
                                     sems[p], add=True)
            return carry

        lax.fori_loop(0, npair, pair, 0)
        for p in range(2):
            for j in range(KB):
                pltpu.make_async_copy(ones_v, deg_sh.at[cidx_v.at[p, j]],
                                      sems[p]).wait()
        plsc.subcore_barrier()
        pltpu.sync_copy(deg_sh.at[pl.ds(sid * rpt, rpt)], zd_v)
        pltpu.sync_copy(zd_v, cnt_hbm.at[pl.ds(cid * acc_rows + sid * rpt, rpt)])

    return deg_kernel(cols2d).reshape(NCORES, acc_rows)


def _sc_scatter(h2d, rows2d, cols2d, acc_rows, n):
    nchunk = cols2d.shape[0] // (NSUB * KB)
    npair = nchunk // 2
    rpt = acc_rows // NSUB
    nz = rpt // 128
    mesh = plsc.VectorSubcoreMesh(core_axis_name="c", subcore_axis_name="s")

    @functools.partial(
        pl.kernel,
        out_type=jax.ShapeDtypeStruct((NCORES, acc_rows, 16), F32),
        mesh=mesh,
        scratch_types=[
            pltpu.VMEM((2, KB, 128), jnp.int32),
            pltpu.VMEM((2, KB, 128), jnp.int32),
            pltpu.VMEM((2, KB, 128, 16), F32),
            pltpu.VMEM_SHARED((acc_rows, 16), F32),
            pltpu.SemaphoreType.DMA,
            pltpu.SemaphoreType.DMA,
            pltpu.SemaphoreType.DMA,
            pltpu.SemaphoreType.DMA,
        ],
        compiler_params=pltpu.CompilerParams(use_tc_tiling_on_sc=False),
    )
    def acc_kernel(h_hbm, rows_hbm, cols_hbm, acc_hbm,
                   ridx_v, cidx_v, rows_v, acc_sh, g0, g1, t0, t1):
        cid = lax.axis_index("c")
        sid = lax.axis_index("s")
        off = cid * n
        tile_base = sid * (nchunk * KB)
        sem_g = (g0, g1)
        sem_s = (t0, t1)

        def zstore(i, carry):
            rows_v[0, 0, i, :] = jnp.zeros((16,), F32)
            return carry

        lax.fori_loop(0, 128, zstore, 0)
        for q in range(nz):
            pltpu.sync_copy(rows_v.at[0, 0],
                            acc_sh.at[pl.ds(sid * rpt + q * 128, 128), :])
        plsc.subcore_barrier()

        def load_and_gather(g, p):
            pltpu.sync_copy(rows_hbm.at[pl.ds(tile_base + g * KB, KB), :],
                            ridx_v.at[p])
            pltpu.sync_copy(cols_hbm.at[pl.ds(tile_base + g * KB, KB), :],
                            cidx_v.at[p])
            for j in range(KB):
                for q in range(8):
                    sl = ridx_v[p, j, pl.ds(q * 16, 16)]
                    ridx_v[p, j, pl.ds(q * 16, 16)] = sl + off
            for j in range(KB):
                pltpu.async_copy(h_hbm.at[ridx_v.at[p, j]], rows_v.at[p, j],
                                 sem_g[p])

        load_and_gather(0, 0)

        def pair(t, carry):
            for p in range(2):
                g = 2 * t + p
                op = 1 - p

                @pl.when(g > 0)
                def _drain():
                    for j in range(KB):
                        pltpu.make_async_copy(rows_v.at[op, j],
                                              acc_sh.at[cidx_v.at[op, j]],
                                              sem_s[op]).wait()

                @pl.when(g + 1 < nchunk)
                def _next():
                    load_and_gather(g + 1, op)

                for j in range(KB):
                    pltpu.make_async_copy(h_hbm.at[ridx_v.at[p, j]],
                                          rows_v.at[p, j], sem_g[p]).wait()
                for j in range(KB):
                    pltpu.async_copy(rows_v.at[p, j],
                                     acc_sh.at[cidx_v.at[p, j]], sem_s[p],
                                     add=True)
            return carry

        lax.fori_loop(0, npair, pair, 0)
        lastp = (nchunk - 1) % 2
        for j in range(KB):
            pltpu.make_async_copy(rows_v.at[lastp, j],
                                  acc_sh.at[cidx_v.at[lastp, j]],
                                  sem_s[lastp]).wait()
        plsc.subcore_barrier()
        for q in range(nz):
            pltpu.sync_copy(acc_sh.at[pl.ds(sid * rpt + q * 128, 128), :],
                            rows_v.at[0, 0])
            pltpu.sync_copy(rows_v.at[0, 0],
                            acc_hbm.at[cid, pl.ds(sid * rpt + q * 128, 128), :])

    return acc_kernel(h2d, rows2d, cols2d)


def _tc_embed(x, W_conv, cnt):
    n, d_in = x.shape
    grid = (pl.cdiv(n, BLK),)

    def body(x_ref, w_ref, cnt_ref, h_ref, dinv_ref):
        deg = cnt_ref[0, :] + cnt_ref[1, :] + 1.0
        dinv = lax.rsqrt(deg)
        h = jnp.dot(x_ref[...], w_ref[...], preferred_element_type=F32)
        hp = h * dinv[:, None]
        h_ref[0] = hp[:, :16]
        h_ref[1] = hp[:, 16:]
        dinv_ref[...] = dinv[:, None]

    return pl.pallas_call(
        body,
        grid=grid,
        in_specs=[
            pl.BlockSpec((BLK, d_in), lambda i: (i, 0)),
            pl.BlockSpec(W_conv.shape, lambda i: (0, 0)),
            pl.BlockSpec((2, BLK), lambda i: (0, i)),
        ],
        out_specs=[
            pl.BlockSpec((2, BLK, 16), lambda i: (0, i, 0)),
            pl.BlockSpec((BLK, 1), lambda i: (i, 0)),
        ],
        out_shape=[
            jax.ShapeDtypeStruct((2, n, 16), F32),
            jax.ShapeDtypeStruct((n, 1), F32),
        ],
    )(x, W_conv, cnt)


def _tc_head(acc, h, dinv, b_conv, W1p, b1p, W2p, b2p):
    n = h.shape[1]
    d_out = W2p.shape[1]
    grid = (pl.cdiv(n, BLK),)

    def body(acc_ref, h_ref, dinv_ref, bc_ref, w1_ref, b1_ref, w2_ref, b2_ref,
             out_ref):
        lo = acc_ref[0] + h_ref[0]
        hi = acc_ref[1] + h_ref[1]
        agg = jnp.concatenate([lo, hi], axis=1) * dinv_ref[...]
        h2 = jnp.maximum(agg + bc_ref[...], 0.0)
        h3 = jnp.maximum(
            jnp.dot(h2, w1_ref[...], preferred_element_type=F32) + b1_ref[...],
            0.0)
        out_ref[...] = (jnp.dot(h3, w2_ref[...], preferred_element_type=F32)
                        + b2_ref[...])

    return pl.pallas_call(
        body,
        grid=grid,
        in_specs=[
            pl.BlockSpec((2, BLK, 16), lambda i: (0, i, 0)),
            pl.BlockSpec((2, BLK, 16), lambda i: (0, i, 0)),
            pl.BlockSpec((BLK, 1), lambda i: (i, 0)),
            pl.BlockSpec(b_conv.shape, lambda i: (0, 0)),
            pl.BlockSpec(W1p.shape, lambda i: (0, 0)),
            pl.BlockSpec(b1p.shape, lambda i: (0, 0)),
            pl.BlockSpec(W2p.shape, lambda i: (0, 0)),
            pl.BlockSpec(b2p.shape, lambda i: (0, 0)),
        ],
        out_specs=pl.BlockSpec((BLK, d_out), lambda i: (i, 0)),
        out_shape=jax.ShapeDtypeStruct((n, d_out), F32),
    )(acc, h, dinv, b_conv, W1p, b1p, W2p, b2p)


def kernel(x, edge_index, W_conv, b_conv, W1, b1, W2, b2):
    n = x.shape[0]
    e = edge_index.shape[1]
    d1 = W_conv.shape[1]
    d2 = W1.shape[1]
    d_out = W2.shape[1]

    unit = NCORES * NSUB * CHUNK * 2
    ep = ((e + unit - 1) // unit) * unit
    acc_rows = ((n + 1 + 1023) // 1024) * 1024

    row_p = jnp.concatenate([edge_index[0],
                             jnp.zeros((ep - e,), jnp.int32)])
    col_p = jnp.concatenate([edge_index[1],
                             jnp.full((ep - e,), n, jnp.int32)])
    rows2d = row_p.reshape(ep // 128, 128)
    cols2d = col_p.reshape(ep // 128, 128)

    W1p = jnp.pad(W1, ((0, 0), (0, 128 - d2)))
    b1p = jnp.pad(b1, (0, 128 - d2)).reshape(1, 128)
    W2p = jnp.pad(W2, ((0, 128 - d2), (0, 0)))
    b2p = b2.reshape(1, d_out)
    bc = b_conv.reshape(1, d1)

    cnt = _sc_degree(cols2d, acc_rows)
    h, dinv = _tc_embed(x, W_conv, cnt)
    h2d = h.reshape(NCORES * n, 16)
    acc = _sc_scatter(h2d, rows2d, cols2d, acc_rows, n)
    out = _tc_head(acc, h, dinv, bc, W1p, b1p, W2p, b2p)
    return out
